# Initial kernel scaffold; baseline (speedup 1.0000x reference)
import jax
import jax.numpy as jnp
from jax.experimental import pallas as pl


def kernel(x, W_val, glb, W_emb, b_emb, a1, b1, W1, c1, a2, b2, W2, c2):
    raise NotImplementedError("write your pallas kernel here")


# speedup vs baseline: 17.5025x; 17.5025x over previous
import numpy as np
import jax
import jax.numpy as jnp
from jax.experimental import pallas as pl

B, T, C = 16, 4096, 64
DM, DFF, G = 128, 256, 16
PCT, LOWK = 0.95, 16
NF = T // 2 + 1
NFP = 2176
BC = B * C
FB = 128
NFB = NFP // FB
RB = 256
NRB = BC // RB


def _build_consts():
    t = np.arange(T, dtype=np.int64)
    f = np.arange(NFP, dtype=np.int64)
    ang = (2.0 * np.pi / T) * ((t[:, None] * f[None, :]) % T).astype(np.float64)
    cos = np.cos(ang)
    sin = np.sin(ang)
    cos[:, 0] = 0.0
    sin[:, 0] = 0.0
    cos[:, NF:] = 0.0
    sin[:, NF:] = 0.0
    w = np.full((1, NFP), 2.0 / T, dtype=np.float64)
    w[0, 0] = 0.0
    w[0, NF - 1] = 1.0 / T
    w[0, NF:] = 0.0
    position = np.arange(C, dtype=np.float32)[:, None]
    div_term = np.exp(np.arange(0, DM, 2, dtype=np.float32) * -(np.log(10000.0) / DM))
    pe = np.zeros((C, DM), dtype=np.float32)
    pe[:, 0::2] = np.sin(position * div_term)
    pe[:, 1::2] = np.cos(position * div_term)
    pe_t = np.tile(pe, (B, 1))
    return (cos.astype(np.float32), sin.astype(np.float32),
            w.astype(np.float32), pe_t.astype(np.float32))


_COS, _SIN, _WVEC, _PE = _build_consts()


def _dft_body(x_ref, wv_ref, cos_ref, sin_ref, xr_ref, xs_ref, wc_ref, ws_ref):
    c = cos_ref[...]
    s = sin_ref[...]
    x = x_ref[...]
    w = wv_ref[...]
    xr_ref[...] = jnp.dot(x, c, preferred_element_type=jnp.float32, precision=jax.lax.Precision.HIGHEST)
    xs_ref[...] = jnp.dot(x, s, preferred_element_type=jnp.float32, precision=jax.lax.Precision.HIGHEST)
    wc_ref[...] = jnp.dot(w, c, preferred_element_type=jnp.float32, precision=jax.lax.Precision.HIGHEST)
    ws_ref[...] = jnp.dot(w, s, preferred_element_type=jnp.float32, precision=jax.lax.Precision.HIGHEST)


def _kat(y, a_ref, b_ref):
    p = a_ref[5:6, :]
    for i in (4, 3, 2, 1, 0):
        p = p * y + a_ref[i:i + 1, :]
    q = b_ref[3:4, :]
    for i in (2, 1, 0):
        q = q * y + b_ref[i:i + 1, :]
    q = q * y
    return p / (1.0 + jnp.abs(q))


def _fuse_body(xr_ref, xs_ref, wc_ref, ws_ref, wvec_ref, pe_ref, glb_ref,
               wembg_ref, wembv_ref, bemb_ref, a1_ref, b1_ref, w1_ref, c1_ref,
               a2_ref, b2_ref, w2_ref, c2_ref, out_ref):
    xr = xr_ref[...]
    xs = xs_ref[...]
    e = xr * xr + xs * xs
    theta = PCT * jnp.sum(e, axis=1, keepdims=True)
    rows = e.shape[0]
    lo0 = jnp.zeros((rows, 1), jnp.int32)
    hi0 = jnp.full((rows, 1), 0x7F7FFFFF, jnp.int32)

    def body(_, carry):
        lo, hi = carry
        mid = lo + jax.lax.shift_right_logical(hi - lo, 1)
        tau = jax.lax.bitcast_convert_type(mid, jnp.float32)
        gt = e > tau
        n = jnp.sum(jnp.where(gt, 1.0, 0.0), axis=1, keepdims=True)
        s = jnp.sum(jnp.where(gt, e, 0.0), axis=1, keepdims=True)
        keep = jnp.logical_or(n < float(LOWK + 1), s < theta)
        return jnp.where(keep, lo, mid + 1), jnp.where(keep, mid, hi)

    lo, _ = jax.lax.fori_loop(0, 31, body, (lo0, hi0))
    ebits = jax.lax.bitcast_convert_type(e, jnp.int32)
    m = jnp.where(ebits >= lo, 1.0, 0.0) * wvec_ref[...]
    zr = xr * m
    zs = xs * m
    dn = (((1,), (1,)), ((), ()))
    ve = (jax.lax.dot_general(zr, wc_ref[...], dn, preferred_element_type=jnp.float32, precision=jax.lax.Precision.HIGHEST)
          + jax.lax.dot_general(zs, ws_ref[...], dn, preferred_element_type=jnp.float32, precision=jax.lax.Precision.HIGHEST)
          + pe_ref[...])
    gb = jax.lax.dot_general(glb_ref[...], wembg_ref[...], dn,
                             preferred_element_type=jnp.float32, precision=jax.lax.Precision.HIGHEST)
    y = (jax.lax.dot_general(ve, wembv_ref[...], dn, preferred_element_type=jnp.float32, precision=jax.lax.Precision.HIGHEST)
         + gb + bemb_ref[...])
    y = _kat(y, a1_ref, b1_ref)
    y = jax.lax.dot_general(y, w1_ref[...], dn, preferred_element_type=jnp.float32, precision=jax.lax.Precision.HIGHEST) + c1_ref[...]
    y = _kat(y, a2_ref, b2_ref)
    out_ref[...] = (jax.lax.dot_general(y, w2_ref[...], dn, preferred_element_type=jnp.float32, precision=jax.lax.Precision.HIGHEST)
                    + c2_ref[...])


def kernel(x, W_val, glb, W_emb, b_emb, a1, b1, W1, c1, a2, b2, W2, c2):
    xT = jnp.transpose(x, (0, 2, 1)).reshape(BC, T)
    cos = jnp.asarray(_COS)
    sin = jnp.asarray(_SIN)
    xr, xs, wc, ws = pl.pallas_call(
        _dft_body,
        grid=(NFB,),
        in_specs=[
            pl.BlockSpec((BC, T), lambda j: (0, 0)),
            pl.BlockSpec((DM, T), lambda j: (0, 0)),
            pl.BlockSpec((T, FB), lambda j: (0, j)),
            pl.BlockSpec((T, FB), lambda j: (0, j)),
        ],
        out_specs=[
            pl.BlockSpec((BC, FB), lambda j: (0, j)),
            pl.BlockSpec((BC, FB), lambda j: (0, j)),
            pl.BlockSpec((DM, FB), lambda j: (0, j)),
            pl.BlockSpec((DM, FB), lambda j: (0, j)),
        ],
        out_shape=[
            jax.ShapeDtypeStruct((BC, NFP), jnp.float32),
            jax.ShapeDtypeStruct((BC, NFP), jnp.float32),
            jax.ShapeDtypeStruct((DM, NFP), jnp.float32),
            jax.ShapeDtypeStruct((DM, NFP), jnp.float32),
        ],
    )(xT, W_val, cos, sin)

    out = pl.pallas_call(
        _fuse_body,
        grid=(NRB,),
        in_specs=[
            pl.BlockSpec((RB, NFP), lambda i: (i, 0)),
            pl.BlockSpec((RB, NFP), lambda i: (i, 0)),
            pl.BlockSpec((DM, NFP), lambda i: (0, 0)),
            pl.BlockSpec((DM, NFP), lambda i: (0, 0)),
            pl.BlockSpec((1, NFP), lambda i: (0, 0)),
            pl.BlockSpec((RB, DM), lambda i: (i, 0)),
            pl.BlockSpec((1, DM), lambda i: (0, 0)),
            pl.BlockSpec((DM, DM), lambda i: (0, 0)),
            pl.BlockSpec((DM, DM), lambda i: (0, 0)),
            pl.BlockSpec((1, DM), lambda i: (0, 0)),
            pl.BlockSpec((6, DM), lambda i: (0, 0)),
            pl.BlockSpec((4, DM), lambda i: (0, 0)),
            pl.BlockSpec((DFF, DM), lambda i: (0, 0)),
            pl.BlockSpec((1, DFF), lambda i: (0, 0)),
            pl.BlockSpec((6, DFF), lambda i: (0, 0)),
            pl.BlockSpec((4, DFF), lambda i: (0, 0)),
            pl.BlockSpec((DM, DFF), lambda i: (0, 0)),
            pl.BlockSpec((1, DM), lambda i: (0, 0)),
        ],
        out_specs=pl.BlockSpec((RB, DM), lambda i: (i, 0)),
        out_shape=jax.ShapeDtypeStruct((BC, DM), jnp.float32),
    )(xr, xs, wc, ws, jnp.asarray(_WVEC), jnp.asarray(_PE),
      glb.reshape(1, DM), W_emb[:, :DM], W_emb[:, DM:], b_emb.reshape(1, DM),
      jnp.repeat(a1, DM // G, axis=0).T, jnp.repeat(b1, DM // G, axis=0).T,
      W1, c1.reshape(1, DFF),
      jnp.repeat(a2, DFF // G, axis=0).T, jnp.repeat(b2, DFF // G, axis=0).T,
      W2, c2.reshape(1, DM))
    return out.reshape(B, C, DM)


# speedup vs baseline: 29.0667x; 1.6607x over previous
import numpy as np
import jax
import jax.numpy as jnp
from jax.experimental import pallas as pl
from jax.experimental.pallas import tpu as pltpu

B, T, C = 16, 4096, 64
DM, DFF, G = 128, 256, 16
PCT, LOWK = 0.95, 16
NF = T // 2 + 1
KH = T // 2
NFP = 2176
NEVB = 9
BC = B * C
FB = 128
NFB = NFP // FB
RB = 256
NRB = BC // RB

_HI = jax.lax.Precision.HIGHEST


def _build_consts():
    fmap = np.zeros(NFP, dtype=np.int64)
    alive = np.zeros(NFP, dtype=bool)
    ncols_even = NEVB * FB
    for j in range(ncols_even):
        f = 2 * j
        if f <= KH:
            fmap[j] = f
            alive[j] = f != 0
    for j in range(ncols_even, NFP):
        f = 2 * (j - ncols_even) + 1
        if f < KH:
            fmap[j] = f
            alive[j] = True
    t = np.arange(KH, dtype=np.int64)
    ang = (2.0 * np.pi / T) * ((t[:, None] * fmap[None, :]) % T).astype(np.float64)
    cos = np.cos(ang) * alive[None, :]
    sin = np.sin(ang) * alive[None, :]
    sin[:, fmap == KH] = 0.0
    w = np.where(alive, 2.0 / T, 0.0)[None, :]
    w[0, fmap == KH] = 1.0 / T
    position = np.arange(C, dtype=np.float32)[:, None]
    div_term = np.exp(np.arange(0, DM, 2, dtype=np.float32) * -(np.log(10000.0) / DM))
    pe = np.zeros((C, DM), dtype=np.float32)
    pe[:, 0::2] = np.sin(position * div_term)
    pe[:, 1::2] = np.cos(position * div_term)
    pe_t = np.tile(pe, (B, 1))
    return (cos.astype(np.float32), sin.astype(np.float32),
            w.astype(np.float32), pe_t.astype(np.float32))


_COSP, _SINP, _WVEC, _PE = _build_consts()


def _dft_body(x_ref, wv_ref, cos_ref, sin_ref,
              xr_ref, xs_ref, wc_ref, ws_ref,
              xu_s, xv_s, wu_s, wv_s):
    j = pl.program_id(0)

    @pl.when(j == 0)
    def _fold():
        x = x_ref[...]
        xu_s[...] = x[:, :KH] + x[:, KH:]
        xv_s[...] = x[:, :KH] - x[:, KH:]
        w = wv_ref[...]
        wu_s[...] = w[:, :KH] + w[:, KH:]
        wv_s[...] = w[:, :KH] - w[:, KH:]

    c = cos_ref[...]
    s = sin_ref[...]

    @pl.when(j < NEVB)
    def _even():
        xr_ref[...] = jnp.dot(xu_s[...], c, preferred_element_type=jnp.float32, precision=_HI)
        xs_ref[...] = jnp.dot(xu_s[...], s, preferred_element_type=jnp.float32, precision=_HI)
        wc_ref[...] = jnp.dot(wu_s[...], c, preferred_element_type=jnp.float32, precision=_HI)
        ws_ref[...] = jnp.dot(wu_s[...], s, preferred_element_type=jnp.float32, precision=_HI)

    @pl.when(j >= NEVB)
    def _odd():
        xr_ref[...] = jnp.dot(xv_s[...], c, preferred_element_type=jnp.float32, precision=_HI)
        xs_ref[...] = jnp.dot(xv_s[...], s, preferred_element_type=jnp.float32, precision=_HI)
        wc_ref[...] = jnp.dot(wv_s[...], c, preferred_element_type=jnp.float32, precision=_HI)
        ws_ref[...] = jnp.dot(wv_s[...], s, preferred_element_type=jnp.float32, precision=_HI)


def _kat(y, a_ref, b_ref):
    p = a_ref[5:6, :]
    for i in (4, 3, 2, 1, 0):
        p = p * y + a_ref[i:i + 1, :]
    q = b_ref[3:4, :]
    for i in (2, 1, 0):
        q = q * y + b_ref[i:i + 1, :]
    q = q * y
    return p / (1.0 + jnp.abs(q))


def _fuse_body(xr_ref, xs_ref, wc_ref, ws_ref, wvec_ref, pe_ref, glb_ref,
               wembg_ref, wembv_ref, bemb_ref, a1_ref, b1_ref, w1_ref, c1_ref,
               a2_ref, b2_ref, w2_ref, c2_ref, out_ref):
    xr = xr_ref[...]
    xs = xs_ref[...]
    e = xr * xr + xs * xs
    theta = PCT * jnp.sum(e, axis=1, keepdims=True)
    rows = e.shape[0]
    lo0 = jnp.zeros((rows, 1), jnp.int32)
    hi0 = jnp.full((rows, 1), 0x7F7FFFFF, jnp.int32)

    def body(_, carry):
        lo, hi = carry
        mid = lo + jax.lax.shift_right_logical(hi - lo, 1)
        tau = jax.lax.bitcast_convert_type(mid, jnp.float32)
        gt = e > tau
        n = jnp.sum(jnp.where(gt, 1.0, 0.0), axis=1, keepdims=True)
        s = jnp.sum(jnp.where(gt, e, 0.0), axis=1, keepdims=True)
        keep = jnp.logical_or(n < float(LOWK + 1), s < theta)
        return jnp.where(keep, lo, mid + 1), jnp.where(keep, mid, hi)

    lo, _ = jax.lax.fori_loop(0, 31, body, (lo0, hi0))
    ebits = jax.lax.bitcast_convert_type(e, jnp.int32)
    m = jnp.where(ebits >= lo, 1.0, 0.0) * wvec_ref[...]
    zr = xr * m
    zs = xs * m
    dn = (((1,), (1,)), ((), ()))
    ve = (jax.lax.dot_general(zr, wc_ref[...], dn, preferred_element_type=jnp.float32, precision=_HI)
          + jax.lax.dot_general(zs, ws_ref[...], dn, preferred_element_type=jnp.float32, precision=_HI)
          + pe_ref[...])
    gb = jax.lax.dot_general(glb_ref[...], wembg_ref[...], dn,
                             preferred_element_type=jnp.float32, precision=_HI)
    y = (jax.lax.dot_general(ve, wembv_ref[...], dn, preferred_element_type=jnp.float32, precision=_HI)
         + gb + bemb_ref[...])
    y = _kat(y, a1_ref, b1_ref)
    y = jax.lax.dot_general(y, w1_ref[...], dn, preferred_element_type=jnp.float32, precision=_HI) + c1_ref[...]
    y = _kat(y, a2_ref, b2_ref)
    out_ref[...] = (jax.lax.dot_general(y, w2_ref[...], dn, preferred_element_type=jnp.float32, precision=_HI)
                    + c2_ref[...])


def kernel(x, W_val, glb, W_emb, b_emb, a1, b1, W1, c1, a2, b2, W2, c2):
    xT = jnp.transpose(x, (0, 2, 1)).reshape(BC, T)
    xr, xs, wc, ws = pl.pallas_call(
        _dft_body,
        grid=(NFB,),
        in_specs=[
            pl.BlockSpec((BC, T), lambda j: (0, 0)),
            pl.BlockSpec((DM, T), lambda j: (0, 0)),
            pl.BlockSpec((KH, FB), lambda j: (0, j)),
            pl.BlockSpec((KH, FB), lambda j: (0, j)),
        ],
        out_specs=[
            pl.BlockSpec((BC, FB), lambda j: (0, j)),
            pl.BlockSpec((BC, FB), lambda j: (0, j)),
            pl.BlockSpec((DM, FB), lambda j: (0, j)),
            pl.BlockSpec((DM, FB), lambda j: (0, j)),
        ],
        out_shape=[
            jax.ShapeDtypeStruct((BC, NFP), jnp.float32),
            jax.ShapeDtypeStruct((BC, NFP), jnp.float32),
            jax.ShapeDtypeStruct((DM, NFP), jnp.float32),
            jax.ShapeDtypeStruct((DM, NFP), jnp.float32),
        ],
        scratch_shapes=[
            pltpu.VMEM((BC, KH), jnp.float32),
            pltpu.VMEM((BC, KH), jnp.float32),
            pltpu.VMEM((DM, KH), jnp.float32),
            pltpu.VMEM((DM, KH), jnp.float32),
        ],
    )(xT, W_val, jnp.asarray(_COSP), jnp.asarray(_SINP))

    out = pl.pallas_call(
        _fuse_body,
        grid=(NRB,),
        in_specs=[
            pl.BlockSpec((RB, NFP), lambda i: (i, 0)),
            pl.BlockSpec((RB, NFP), lambda i: (i, 0)),
            pl.BlockSpec((DM, NFP), lambda i: (0, 0)),
            pl.BlockSpec((DM, NFP), lambda i: (0, 0)),
            pl.BlockSpec((1, NFP), lambda i: (0, 0)),
            pl.BlockSpec((RB, DM), lambda i: (i, 0)),
            pl.BlockSpec((1, DM), lambda i: (0, 0)),
            pl.BlockSpec((DM, DM), lambda i: (0, 0)),
            pl.BlockSpec((DM, DM), lambda i: (0, 0)),
            pl.BlockSpec((1, DM), lambda i: (0, 0)),
            pl.BlockSpec((6, DM), lambda i: (0, 0)),
            pl.BlockSpec((4, DM), lambda i: (0, 0)),
            pl.BlockSpec((DFF, DM), lambda i: (0, 0)),
            pl.BlockSpec((1, DFF), lambda i: (0, 0)),
            pl.BlockSpec((6, DFF), lambda i: (0, 0)),
            pl.BlockSpec((4, DFF), lambda i: (0, 0)),
            pl.BlockSpec((DM, DFF), lambda i: (0, 0)),
            pl.BlockSpec((1, DM), lambda i: (0, 0)),
        ],
        out_specs=pl.BlockSpec((RB, DM), lambda i: (i, 0)),
        out_shape=jax.ShapeDtypeStruct((BC, DM), jnp.float32),
    )(xr, xs, wc, ws, jnp.asarray(_WVEC), jnp.asarray(_PE),
      glb.reshape(1, DM), W_emb[:, :DM], W_emb[:, DM:], b_emb.reshape(1, DM),
      jnp.repeat(a1, DM // G, axis=0).T, jnp.repeat(b1, DM // G, axis=0).T,
      W1, c1.reshape(1, DFF),
      jnp.repeat(a2, DFF // G, axis=0).T, jnp.repeat(b2, DFF // G, axis=0).T,
      W2, c2.reshape(1, DM))
    return out.reshape(B, C, DM)


# speedup vs baseline: 41.4166x; 1.4249x over previous
import numpy as np
import jax
import jax.numpy as jnp
from jax.experimental import pallas as pl
from jax.experimental.pallas import tpu as pltpu

B, T, C = 16, 4096, 64
DM, DFF, G = 128, 256, 16
PCT, LOWK = 0.95, 16
NF = T // 2 + 1
KH = T // 2
NFP = 2176
NEVB = 9
BC = B * C
FB = 128
NFB = NFP // FB
RB = 256
NRB = BC // RB

_HI = jax.lax.Precision.HIGHEST


def _build_consts():
    fmap = np.zeros(NFP, dtype=np.int64)
    alive = np.zeros(NFP, dtype=bool)
    ncols_even = NEVB * FB
    for j in range(ncols_even):
        f = 2 * j
        if f <= KH:
            fmap[j] = f
            alive[j] = f != 0
    for j in range(ncols_even, NFP):
        f = 2 * (j - ncols_even) + 1
        if f < KH:
            fmap[j] = f
            alive[j] = True
    t = np.arange(KH, dtype=np.int64)
    ang = (2.0 * np.pi / T) * ((t[:, None] * fmap[None, :]) % T).astype(np.float64)
    cos = np.cos(ang) * alive[None, :]
    sin = np.sin(ang) * alive[None, :]
    sin[:, fmap == KH] = 0.0
    cs = np.zeros((KH, 2 * NFP))
    for j in range(NFB):
        cs[:, 2 * FB * j:2 * FB * j + FB] = cos[:, FB * j:FB * (j + 1)]
        cs[:, 2 * FB * j + FB:2 * FB * (j + 1)] = sin[:, FB * j:FB * (j + 1)]
    w = np.where(alive, 2.0 / T, 0.0)[None, :]
    w[0, fmap == KH] = 1.0 / T
    position = np.arange(C, dtype=np.float32)[:, None]
    div_term = np.exp(np.arange(0, DM, 2, dtype=np.float32) * -(np.log(10000.0) / DM))
    pe = np.zeros((C, DM), dtype=np.float32)
    pe[:, 0::2] = np.sin(position * div_term)
    pe[:, 1::2] = np.cos(position * div_term)
    pe_t = np.tile(pe, (B, 1))
    return (cs.astype(np.float32),
            w.astype(np.float32), pe_t.astype(np.float32))


_CS, _WVEC, _PE = _build_consts()


def _dft_body(x_ref, wv_ref, cs_ref,
              xr_ref, xs_ref, wc_ref, ws_ref,
              zu_s, zv_s):
    j = pl.program_id(0)

    @pl.when(j == 0)
    def _fold():
        x = x_ref[...]
        zu_s[:BC, :] = x[:, :KH] + x[:, KH:]
        zv_s[:BC, :] = x[:, :KH] - x[:, KH:]
        w = wv_ref[...]
        zu_s[BC:, :] = w[:, :KH] + w[:, KH:]
        zv_s[BC:, :] = w[:, :KH] - w[:, KH:]

    cs = cs_ref[...]

    @pl.when(j < NEVB)
    def _even():
        z = jnp.dot(zu_s[...], cs, preferred_element_type=jnp.float32, precision=_HI)
        xr_ref[...] = z[:BC, :FB]
        xs_ref[...] = z[:BC, FB:]
        wc_ref[...] = z[BC:, :FB]
        ws_ref[...] = z[BC:, FB:]

    @pl.when(j >= NEVB)
    def _odd():
        z = jnp.dot(zv_s[...], cs, preferred_element_type=jnp.float32, precision=_HI)
        xr_ref[...] = z[:BC, :FB]
        xs_ref[...] = z[:BC, FB:]
        wc_ref[...] = z[BC:, :FB]
        ws_ref[...] = z[BC:, FB:]


def _kat(y, a_ref, b_ref):
    p = a_ref[5:6, :]
    for i in (4, 3, 2, 1, 0):
        p = p * y + a_ref[i:i + 1, :]
    q = b_ref[3:4, :]
    for i in (2, 1, 0):
        q = q * y + b_ref[i:i + 1, :]
    q = q * y
    return p / (1.0 + jnp.abs(q))


def _fuse_body(xr_ref, xs_ref, wc_ref, ws_ref, wvec_ref, pe_ref, glb_ref,
               wembg_ref, wembv_ref, bemb_ref, a1_ref, b1_ref, w1_ref, c1_ref,
               a2_ref, b2_ref, w2_ref, c2_ref, out_ref):
    xr = xr_ref[...]
    xs = xs_ref[...]
    e = xr * xr + xs * xs
    theta = PCT * jnp.sum(e, axis=1, keepdims=True)
    rows = e.shape[0]
    lo0 = jnp.zeros((rows, 1), jnp.int32)
    hi0 = jnp.full((rows, 1), 0x7F7FFFFF, jnp.int32)

    def body(_, carry):
        lo, hi = carry
        mid = lo + jax.lax.shift_right_logical(hi - lo, 1)
        tau = jax.lax.bitcast_convert_type(mid, jnp.float32)
        gt = e > tau
        n = jnp.sum(jnp.where(gt, 1.0, 0.0), axis=1, keepdims=True)
        s = jnp.sum(jnp.where(gt, e, 0.0), axis=1, keepdims=True)
        keep = jnp.logical_or(n < float(LOWK + 1), s < theta)
        return jnp.where(keep, lo, mid + 1), jnp.where(keep, mid, hi)

    lo, _ = jax.lax.fori_loop(0, 31, body, (lo0, hi0))
    ebits = jax.lax.bitcast_convert_type(e, jnp.int32)
    m = jnp.where(ebits >= lo, 1.0, 0.0) * wvec_ref[...]
    zr = xr * m
    zs = xs * m
    dn = (((1,), (1,)), ((), ()))
    ve = (jax.lax.dot_general(zr, wc_ref[...], dn, preferred_element_type=jnp.float32, precision=_HI)
          + jax.lax.dot_general(zs, ws_ref[...], dn, preferred_element_type=jnp.float32, precision=_HI)
          + pe_ref[...])
    gb = jax.lax.dot_general(glb_ref[...], wembg_ref[...], dn,
                             preferred_element_type=jnp.float32, precision=_HI)
    y = (jax.lax.dot_general(ve, wembv_ref[...], dn, preferred_element_type=jnp.float32, precision=_HI)
         + gb + bemb_ref[...])
    y = _kat(y, a1_ref, b1_ref)
    y = jax.lax.dot_general(y, w1_ref[...], dn, preferred_element_type=jnp.float32, precision=_HI) + c1_ref[...]
    y = _kat(y, a2_ref, b2_ref)
    out_ref[...] = (jax.lax.dot_general(y, w2_ref[...], dn, preferred_element_type=jnp.float32, precision=_HI)
                    + c2_ref[...])


def kernel(x, W_val, glb, W_emb, b_emb, a1, b1, W1, c1, a2, b2, W2, c2):
    xT = jnp.transpose(x, (0, 2, 1)).reshape(BC, T)
    xr, xs, wc, ws = pl.pallas_call(
        _dft_body,
        grid=(NFB,),
        in_specs=[
            pl.BlockSpec((BC, T), lambda j: (0, 0)),
            pl.BlockSpec((DM, T), lambda j: (0, 0)),
            pl.BlockSpec((KH, 2 * FB), lambda j: (0, j)),
        ],
        out_specs=[
            pl.BlockSpec((BC, FB), lambda j: (0, j)),
            pl.BlockSpec((BC, FB), lambda j: (0, j)),
            pl.BlockSpec((DM, FB), lambda j: (0, j)),
            pl.BlockSpec((DM, FB), lambda j: (0, j)),
        ],
        out_shape=[
            jax.ShapeDtypeStruct((BC, NFP), jnp.float32),
            jax.ShapeDtypeStruct((BC, NFP), jnp.float32),
            jax.ShapeDtypeStruct((DM, NFP), jnp.float32),
            jax.ShapeDtypeStruct((DM, NFP), jnp.float32),
        ],
        scratch_shapes=[
            pltpu.VMEM((BC + DM, KH), jnp.float32),
            pltpu.VMEM((BC + DM, KH), jnp.float32),
        ],
    )(xT, W_val, jnp.asarray(_CS))

    out = pl.pallas_call(
        _fuse_body,
        grid=(NRB,),
        in_specs=[
            pl.BlockSpec((RB, NFP), lambda i: (i, 0)),
            pl.BlockSpec((RB, NFP), lambda i: (i, 0)),
            pl.BlockSpec((DM, NFP), lambda i: (0, 0)),
            pl.BlockSpec((DM, NFP), lambda i: (0, 0)),
            pl.BlockSpec((1, NFP), lambda i: (0, 0)),
            pl.BlockSpec((RB, DM), lambda i: (i, 0)),
            pl.BlockSpec((1, DM), lambda i: (0, 0)),
            pl.BlockSpec((DM, DM), lambda i: (0, 0)),
            pl.BlockSpec((DM, DM), lambda i: (0, 0)),
            pl.BlockSpec((1, DM), lambda i: (0, 0)),
            pl.BlockSpec((6, DM), lambda i: (0, 0)),
            pl.BlockSpec((4, DM), lambda i: (0, 0)),
            pl.BlockSpec((DFF, DM), lambda i: (0, 0)),
            pl.BlockSpec((1, DFF), lambda i: (0, 0)),
            pl.BlockSpec((6, DFF), lambda i: (0, 0)),
            pl.BlockSpec((4, DFF), lambda i: (0, 0)),
            pl.BlockSpec((DM, DFF), lambda i: (0, 0)),
            pl.BlockSpec((1, DM), lambda i: (0, 0)),
        ],
        out_specs=pl.BlockSpec((RB, DM), lambda i: (i, 0)),
        out_shape=jax.ShapeDtypeStruct((BC, DM), jnp.float32),
    )(xr, xs, wc, ws, jnp.asarray(_WVEC), jnp.asarray(_PE),
      glb.reshape(1, DM), W_emb[:, :DM], W_emb[:, DM:], b_emb.reshape(1, DM),
      jnp.repeat(a1, DM // G, axis=0).T, jnp.repeat(b1, DM // G, axis=0).T,
      W1, c1.reshape(1, DFF),
      jnp.repeat(a2, DFF // G, axis=0).T, jnp.repeat(b2, DFF // G, axis=0).T,
      W2, c2.reshape(1, DM))
    return out.reshape(B, C, DM)


# speedup vs baseline: 48.5728x; 1.1728x over previous
import numpy as np
import jax
import jax.numpy as jnp
from jax.experimental import pallas as pl
from jax.experimental.pallas import tpu as pltpu

B, T, C = 16, 4096, 64
DM, DFF, G = 128, 256, 16
PCT, LOWK = 0.95, 16
NF = T // 2 + 1
KH = T // 2
KQ = T // 4
NFP = 2176
NA, NB, NC = 5, 4, 8
BC = B * C
FB = 128
NFB = NFP // FB
RB = 256
NRB = BC // RB

_HI = jax.lax.Precision.HIGHEST
_DFT_PREC = jax.lax.Precision.HIGHEST


def _build_consts():
    fmap = np.zeros(NFP, dtype=np.int64)
    alive = np.zeros(NFP, dtype=bool)
    for j in range(NA * FB):
        f = 4 * j
        if f <= KH:
            fmap[j] = f
            alive[j] = f != 0
    for m in range(NB * FB):
        fmap[NA * FB + m] = 4 * m + 2
        alive[NA * FB + m] = True
    for m in range(NC * FB):
        fmap[(NA + NB) * FB + m] = 2 * m + 1
        alive[(NA + NB) * FB + m] = True

    def _tab(k, period, freqs, live):
        tt = np.arange(k, dtype=np.int64)
        ang = (2.0 * np.pi / period) * ((tt[:, None] * freqs[None, :]) % period).astype(np.float64)
        c = np.cos(ang) * live[None, :]
        s = np.sin(ang) * live[None, :]
        return c, s

    cA, sA = _tab(KQ, KQ, fmap[:NA * FB] // 4, alive[:NA * FB])
    sA[:, fmap[:NA * FB] == KH] = 0.0
    cB, sB = _tab(KQ, KH, fmap[NA * FB:(NA + NB) * FB] // 2, alive[NA * FB:(NA + NB) * FB])
    cC, sC = _tab(KH, T, fmap[(NA + NB) * FB:], alive[(NA + NB) * FB:])
    cs_e = np.zeros((KQ, 2 * FB * (NA + NB)))
    for k in range(NA + NB):
        c = cA[:, FB * k:FB * (k + 1)] if k < NA else cB[:, FB * (k - NA):FB * (k - NA + 1)]
        s = sA[:, FB * k:FB * (k + 1)] if k < NA else sB[:, FB * (k - NA):FB * (k - NA + 1)]
        cs_e[:, 2 * FB * k:2 * FB * k + FB] = c
        cs_e[:, 2 * FB * k + FB:2 * FB * (k + 1)] = s
    cs_c = np.zeros((KH, 2 * FB * NC))
    for k in range(NC):
        cs_c[:, 2 * FB * k:2 * FB * k + FB] = cC[:, FB * k:FB * (k + 1)]
        cs_c[:, 2 * FB * k + FB:2 * FB * (k + 1)] = sC[:, FB * k:FB * (k + 1)]
    w = np.where(alive, 2.0 / T, 0.0)[None, :]
    w[0, fmap == KH] = 1.0 / T
    position = np.arange(C, dtype=np.float32)[:, None]
    div_term = np.exp(np.arange(0, DM, 2, dtype=np.float32) * -(np.log(10000.0) / DM))
    pe = np.zeros((C, DM), dtype=np.float32)
    pe[:, 0::2] = np.sin(position * div_term)
    pe[:, 1::2] = np.cos(position * div_term)
    pe_t = np.tile(pe, (B, 1))
    return (cs_e.astype(np.float32), cs_c.astype(np.float32),
            w.astype(np.float32), pe_t.astype(np.float32))


_CSE, _CSC, _WVEC, _PE = _build_consts()


def _dft_body(x_ref, wv_ref, cse_ref, csc_ref,
              xr_ref, xs_ref, wc_ref, ws_ref,
              zuu_s, zuv_s, zv_s):
    j = pl.program_id(0)

    @pl.when(j == 0)
    def _fold():
        x = x_ref[...]
        u = x[:, :KH] + x[:, KH:]
        zv_s[:BC, :] = x[:, :KH] - x[:, KH:]
        zuu_s[:BC, :] = u[:, :KQ] + u[:, KQ:]
        zuv_s[:BC, :] = u[:, :KQ] - u[:, KQ:]
        w = wv_ref[...]
        wu = w[:, :KH] + w[:, KH:]
        zv_s[BC:, :] = w[:, :KH] - w[:, KH:]
        zuu_s[BC:, :] = wu[:, :KQ] + wu[:, KQ:]
        zuv_s[BC:, :] = wu[:, :KQ] - wu[:, KQ:]

    def _emit(z):
        xr_ref[...] = z[:BC, :FB]
        xs_ref[...] = z[:BC, FB:]
        wc_ref[...] = z[BC:, :FB]
        ws_ref[...] = z[BC:, FB:]

    @pl.when(j < NA)
    def _a():
        _emit(jnp.dot(zuu_s[...], cse_ref[...], preferred_element_type=jnp.float32, precision=_HI))

    @pl.when(jnp.logical_and(j >= NA, j < NA + NB))
    def _b():
        _emit(jnp.dot(zuv_s[...], cse_ref[...], preferred_element_type=jnp.float32, precision=_HI))

    @pl.when(j >= NA + NB)
    def _c():
        _emit(jnp.dot(zv_s[...], csc_ref[...], preferred_element_type=jnp.float32, precision=_HI))


def _kat(y, a_ref, b_ref):
    p = a_ref[5:6, :]
    for i in (4, 3, 2, 1, 0):
        p = p * y + a_ref[i:i + 1, :]
    q = b_ref[3:4, :]
    for i in (2, 1, 0):
        q = q * y + b_ref[i:i + 1, :]
    q = q * y
    return p / (1.0 + jnp.abs(q))


def _fuse_body(xr_ref, xs_ref, wc_ref, ws_ref, wvec_ref, pe_ref, glb_ref,
               wembg_ref, wembv_ref, bemb_ref, a1_ref, b1_ref, w1_ref, c1_ref,
               a2_ref, b2_ref, w2_ref, c2_ref, out_ref):
    xr = xr_ref[...]
    xs = xs_ref[...]
    e = xr * xr + xs * xs
    theta = PCT * jnp.sum(e, axis=1, keepdims=True)
    rows = e.shape[0]
    lo0 = jnp.zeros((rows, 1), jnp.int32)
    hi0 = jnp.full((rows, 1), 0x7F7FFFFF, jnp.int32)

    def body(_, carry):
        lo, hi = carry
        mid = lo + jax.lax.shift_right_logical(hi - lo, 1)
        tau = jax.lax.bitcast_convert_type(mid, jnp.float32)
        gt = e > tau
        n = jnp.sum(jnp.where(gt, 1.0, 0.0), axis=1, keepdims=True)
        s = jnp.sum(jnp.where(gt, e, 0.0), axis=1, keepdims=True)
        keep = jnp.logical_or(n < float(LOWK + 1), s < theta)
        return jnp.where(keep, lo, mid + 1), jnp.where(keep, mid, hi)

    lo, _ = jax.lax.fori_loop(0, 31, body, (lo0, hi0))
    ebits = jax.lax.bitcast_convert_type(e, jnp.int32)
    m = jnp.where(ebits >= lo, 1.0, 0.0) * wvec_ref[...]
    zr = xr * m
    zs = xs * m
    dn = (((1,), (1,)), ((), ()))
    ve = (jax.lax.dot_general(zr, wc_ref[...], dn, preferred_element_type=jnp.float32, precision=_HI)
          + jax.lax.dot_general(zs, ws_ref[...], dn, preferred_element_type=jnp.float32, precision=_HI)
          + pe_ref[...])
    gb = jax.lax.dot_general(glb_ref[...], wembg_ref[...], dn,
                             preferred_element_type=jnp.float32, precision=_HI)
    y = (jax.lax.dot_general(ve, wembv_ref[...], dn, preferred_element_type=jnp.float32, precision=_HI)
         + gb + bemb_ref[...])
    y = _kat(y, a1_ref, b1_ref)
    y = jax.lax.dot_general(y, w1_ref[...], dn, preferred_element_type=jnp.float32, precision=_HI) + c1_ref[...]
    y = _kat(y, a2_ref, b2_ref)
    out_ref[...] = (jax.lax.dot_general(y, w2_ref[...], dn, preferred_element_type=jnp.float32, precision=_HI)
                    + c2_ref[...])


def kernel(x, W_val, glb, W_emb, b_emb, a1, b1, W1, c1, a2, b2, W2, c2):
    xT = jnp.transpose(x, (0, 2, 1)).reshape(BC, T)
    xr, xs, wc, ws = pl.pallas_call(
        _dft_body,
        grid=(NFB,),
        in_specs=[
            pl.BlockSpec((BC, T), lambda j: (0, 0)),
            pl.BlockSpec((DM, T), lambda j: (0, 0)),
            pl.BlockSpec((KQ, 2 * FB), lambda j: (0, jnp.minimum(j, NA + NB - 1))),
            pl.BlockSpec((KH, 2 * FB), lambda j: (0, jnp.maximum(j - (NA + NB), 0))),
        ],
        out_specs=[
            pl.BlockSpec((BC, FB), lambda j: (0, j)),
            pl.BlockSpec((BC, FB), lambda j: (0, j)),
            pl.BlockSpec((DM, FB), lambda j: (0, j)),
            pl.BlockSpec((DM, FB), lambda j: (0, j)),
        ],
        out_shape=[
            jax.ShapeDtypeStruct((BC, NFP), jnp.float32),
            jax.ShapeDtypeStruct((BC, NFP), jnp.float32),
            jax.ShapeDtypeStruct((DM, NFP), jnp.float32),
            jax.ShapeDtypeStruct((DM, NFP), jnp.float32),
        ],
        scratch_shapes=[
            pltpu.VMEM((BC + DM, KQ), jnp.float32),
            pltpu.VMEM((BC + DM, KQ), jnp.float32),
            pltpu.VMEM((BC + DM, KH), jnp.float32),
        ],
    )(xT, W_val, jnp.asarray(_CSE), jnp.asarray(_CSC))

    out = pl.pallas_call(
        _fuse_body,
        grid=(NRB,),
        in_specs=[
            pl.BlockSpec((RB, NFP), lambda i: (i, 0)),
            pl.BlockSpec((RB, NFP), lambda i: (i, 0)),
            pl.BlockSpec((DM, NFP), lambda i: (0, 0)),
            pl.BlockSpec((DM, NFP), lambda i: (0, 0)),
            pl.BlockSpec((1, NFP), lambda i: (0, 0)),
            pl.BlockSpec((RB, DM), lambda i: (i, 0)),
            pl.BlockSpec((1, DM), lambda i: (0, 0)),
            pl.BlockSpec((DM, DM), lambda i: (0, 0)),
            pl.BlockSpec((DM, DM), lambda i: (0, 0)),
            pl.BlockSpec((1, DM), lambda i: (0, 0)),
            pl.BlockSpec((6, DM), lambda i: (0, 0)),
            pl.BlockSpec((4, DM), lambda i: (0, 0)),
            pl.BlockSpec((DFF, DM), lambda i: (0, 0)),
            pl.BlockSpec((1, DFF), lambda i: (0, 0)),
            pl.BlockSpec((6, DFF), lambda i: (0, 0)),
            pl.BlockSpec((4, DFF), lambda i: (0, 0)),
            pl.BlockSpec((DM, DFF), lambda i: (0, 0)),
            pl.BlockSpec((1, DM), lambda i: (0, 0)),
        ],
        out_specs=pl.BlockSpec((RB, DM), lambda i: (i, 0)),
        out_shape=jax.ShapeDtypeStruct((BC, DM), jnp.float32),
    )(xr, xs, wc, ws, jnp.asarray(_WVEC), jnp.asarray(_PE),
      glb.reshape(1, DM), W_emb[:, :DM], W_emb[:, DM:], b_emb.reshape(1, DM),
      jnp.repeat(a1, DM // G, axis=0).T, jnp.repeat(b1, DM // G, axis=0).T,
      W1, c1.reshape(1, DFF),
      jnp.repeat(a2, DFF // G, axis=0).T, jnp.repeat(b2, DFF // G, axis=0).T,
      W2, c2.reshape(1, DM))
    return out.reshape(B, C, DM)


# speedup vs baseline: 50.7022x; 1.0438x over previous
import numpy as np
import jax
import jax.numpy as jnp
from jax.experimental import pallas as pl
from jax.experimental.pallas import tpu as pltpu

B, T, C = 16, 4096, 64
DM, DFF, G = 128, 256, 16
PCT, LOWK = 0.95, 16
NF = T // 2 + 1
KH = T // 2
KQ = T // 4
KE = T // 8
NFP = 2176
NA0, NA1, NB, NC = 3, 2, 4, 8
NA = NA0 + NA1
BC = B * C
FB = 128
NFB = NFP // FB
RB = 256
NRB = BC // RB

_HI = jax.lax.Precision.HIGHEST
_DFT_PREC = jax.lax.Precision.HIGHEST


def _build_consts():
    fmap = np.zeros(NFP, dtype=np.int64)
    alive = np.zeros(NFP, dtype=bool)
    for n in range(NA0 * FB):
        f = 8 * n
        if f <= KH:
            fmap[n] = f
            alive[n] = f != 0
    for n in range(NA1 * FB):
        fmap[NA0 * FB + n] = 8 * n + 4
        alive[NA0 * FB + n] = True
    for m in range(NB * FB):
        fmap[NA * FB + m] = 4 * m + 2
        alive[NA * FB + m] = True
    for m in range(NC * FB):
        fmap[(NA + NB) * FB + m] = 2 * m + 1
        alive[(NA + NB) * FB + m] = True

    def _tab(k, period, freqs, live):
        tt = np.arange(k, dtype=np.int64)
        ang = (2.0 * np.pi / period) * ((tt[:, None] * freqs[None, :]) % period).astype(np.float64)
        c = np.cos(ang) * live[None, :]
        s = np.sin(ang) * live[None, :]
        return c, s

    cA0, sA0 = _tab(KE, KE, fmap[:NA0 * FB] // 8, alive[:NA0 * FB])
    sA0[:, fmap[:NA0 * FB] == KH] = 0.0
    cA1, sA1 = _tab(KE, KQ, fmap[NA0 * FB:NA * FB] // 4, alive[NA0 * FB:NA * FB])
    cB, sB = _tab(KQ, KH, fmap[NA * FB:(NA + NB) * FB] // 2, alive[NA * FB:(NA + NB) * FB])
    cC, sC = _tab(KH, T, fmap[(NA + NB) * FB:], alive[(NA + NB) * FB:])

    def _ilv(c, s):
        k, ncol = c.shape
        out = np.zeros((k, 2 * ncol))
        for blk in range(ncol // FB):
            out[:, 2 * FB * blk:2 * FB * blk + FB] = c[:, FB * blk:FB * (blk + 1)]
            out[:, 2 * FB * blk + FB:2 * FB * (blk + 1)] = s[:, FB * blk:FB * (blk + 1)]
        return out

    cs_a = _ilv(np.concatenate([cA0, cA1], 1), np.concatenate([sA0, sA1], 1))
    cs_b = _ilv(cB, sB)
    cs_c = _ilv(cC, sC)
    w = np.where(alive, 2.0 / T, 0.0)[None, :]
    w[0, fmap == KH] = 1.0 / T
    position = np.arange(C, dtype=np.float32)[:, None]
    div_term = np.exp(np.arange(0, DM, 2, dtype=np.float32) * -(np.log(10000.0) / DM))
    pe = np.zeros((C, DM), dtype=np.float32)
    pe[:, 0::2] = np.sin(position * div_term)
    pe[:, 1::2] = np.cos(position * div_term)
    pe_t = np.tile(pe, (B, 1))
    return (cs_a.astype(np.float32), cs_b.astype(np.float32),
            cs_c.astype(np.float32),
            w.astype(np.float32), pe_t.astype(np.float32))


_CSA, _CSB, _CSC, _WVEC, _PE = _build_consts()


def _dft_body(x_ref, wv_ref, csa_ref, csb_ref, csc_ref,
              xr_ref, xs_ref, wc_ref, ws_ref,
              zuuu_s, zuuv_s, zuv_s, zv_s):
    j = pl.program_id(0)

    @pl.when(j == 0)
    def _fold():
        x = x_ref[...]
        u = x[:, :KH] + x[:, KH:]
        uu = u[:, :KQ] + u[:, KQ:]
        zuv_s[:BC, :] = u[:, :KQ] - u[:, KQ:]
        zuuu_s[:BC, :] = uu[:, :KE] + uu[:, KE:]
        zuuv_s[:BC, :] = uu[:, :KE] - uu[:, KE:]
        w = wv_ref[...]
        wu = w[:, :KH] + w[:, KH:]
        wuu = wu[:, :KQ] + wu[:, KQ:]
        zuv_s[BC:, :] = wu[:, :KQ] - wu[:, KQ:]
        zuuu_s[BC:, :] = wuu[:, :KE] + wuu[:, KE:]
        zuuv_s[BC:, :] = wuu[:, :KE] - wuu[:, KE:]

    @pl.when(j == 1)
    def _fold_v():
        x = x_ref[...]
        zv_s[:BC, :] = x[:, :KH] - x[:, KH:]
        w = wv_ref[...]
        zv_s[BC:, :] = w[:, :KH] - w[:, KH:]

    def _emit(z):
        xr_ref[...] = z[:BC, :FB]
        xs_ref[...] = z[:BC, FB:]
        wc_ref[...] = z[BC:, :FB]
        ws_ref[...] = z[BC:, FB:]

    @pl.when(j < NA0)
    def _a0():
        _emit(jnp.dot(zuuu_s[...], csa_ref[...], preferred_element_type=jnp.float32, precision=_HI))

    @pl.when(jnp.logical_and(j >= NA0, j < NA))
    def _a1():
        _emit(jnp.dot(zuuv_s[...], csa_ref[...], preferred_element_type=jnp.float32, precision=_HI))

    @pl.when(jnp.logical_and(j >= NA, j < NA + NB))
    def _b():
        _emit(jnp.dot(zuv_s[...], csb_ref[...], preferred_element_type=jnp.float32, precision=_HI))

    @pl.when(j >= NA + NB)
    def _c():
        _emit(jnp.dot(zv_s[...], csc_ref[...], preferred_element_type=jnp.float32, precision=_HI))


def _kat(y, a_ref, b_ref):
    p = a_ref[5:6, :]
    for i in (4, 3, 2, 1, 0):
        p = p * y + a_ref[i:i + 1, :]
    q = b_ref[3:4, :]
    for i in (2, 1, 0):
        q = q * y + b_ref[i:i + 1, :]
    q = q * y
    return p / (1.0 + jnp.abs(q))


def _fuse_body(xr_ref, xs_ref, wc_ref, ws_ref, wvec_ref, pe_ref, glb_ref,
               wembg_ref, wembv_ref, bemb_ref, a1_ref, b1_ref, w1_ref, c1_ref,
               a2_ref, b2_ref, w2_ref, c2_ref, out_ref):
    xr = xr_ref[...]
    xs = xs_ref[...]
    e = xr * xr + xs * xs
    theta = PCT * jnp.sum(e, axis=1, keepdims=True)
    rows = e.shape[0]
    lo0 = jnp.zeros((rows, 1), jnp.int32)
    hi0 = jnp.full((rows, 1), 0x7F7FFFFF, jnp.int32)

    def body(_, carry):
        lo, hi = carry
        mid = lo + jax.lax.shift_right_logical(hi - lo, 1)
        tau = jax.lax.bitcast_convert_type(mid, jnp.float32)
        gt = e > tau
        n = jnp.sum(jnp.where(gt, 1.0, 0.0), axis=1, keepdims=True)
        s = jnp.sum(jnp.where(gt, e, 0.0), axis=1, keepdims=True)
        keep = jnp.logical_or(n < float(LOWK + 1), s < theta)
        return jnp.where(keep, lo, mid + 1), jnp.where(keep, mid, hi)

    lo, _ = jax.lax.fori_loop(0, 31, body, (lo0, hi0))
    ebits = jax.lax.bitcast_convert_type(e, jnp.int32)
    m = jnp.where(ebits >= lo, 1.0, 0.0) * wvec_ref[...]
    zr = xr * m
    zs = xs * m
    dn = (((1,), (1,)), ((), ()))
    ve = (jax.lax.dot_general(zr, wc_ref[...], dn, preferred_element_type=jnp.float32, precision=_HI)
          + jax.lax.dot_general(zs, ws_ref[...], dn, preferred_element_type=jnp.float32, precision=_HI)
          + pe_ref[...])
    gb = jax.lax.dot_general(glb_ref[...], wembg_ref[...], dn,
                             preferred_element_type=jnp.float32, precision=_HI)
    y = (jax.lax.dot_general(ve, wembv_ref[...], dn, preferred_element_type=jnp.float32, precision=_HI)
         + gb + bemb_ref[...])
    y = _kat(y, a1_ref, b1_ref)
    y = jax.lax.dot_general(y, w1_ref[...], dn, preferred_element_type=jnp.float32, precision=_HI) + c1_ref[...]
    y = _kat(y, a2_ref, b2_ref)
    out_ref[...] = (jax.lax.dot_general(y, w2_ref[...], dn, preferred_element_type=jnp.float32, precision=_HI)
                    + c2_ref[...])


def kernel(x, W_val, glb, W_emb, b_emb, a1, b1, W1, c1, a2, b2, W2, c2):
    xT = jnp.transpose(x, (0, 2, 1)).reshape(BC, T)
    xr, xs, wc, ws = pl.pallas_call(
        _dft_body,
        grid=(NFB,),
        in_specs=[
            pl.BlockSpec((BC, T), lambda j: (0, 0)),
            pl.BlockSpec((DM, T), lambda j: (0, 0)),
            pl.BlockSpec((KE, 2 * FB), lambda j: (0, jnp.minimum(j, NA - 1))),
            pl.BlockSpec((KQ, 2 * FB), lambda j: (0, jnp.clip(j - NA, 0, NB - 1))),
            pl.BlockSpec((KH, 2 * FB), lambda j: (0, jnp.maximum(j - (NA + NB), 0))),
        ],
        out_specs=[
            pl.BlockSpec((BC, FB), lambda j: (0, j)),
            pl.BlockSpec((BC, FB), lambda j: (0, j)),
            pl.BlockSpec((DM, FB), lambda j: (0, j)),
            pl.BlockSpec((DM, FB), lambda j: (0, j)),
        ],
        out_shape=[
            jax.ShapeDtypeStruct((BC, NFP), jnp.float32),
            jax.ShapeDtypeStruct((BC, NFP), jnp.float32),
            jax.ShapeDtypeStruct((DM, NFP), jnp.float32),
            jax.ShapeDtypeStruct((DM, NFP), jnp.float32),
        ],
        scratch_shapes=[
            pltpu.VMEM((BC + DM, KE), jnp.float32),
            pltpu.VMEM((BC + DM, KE), jnp.float32),
            pltpu.VMEM((BC + DM, KQ), jnp.float32),
            pltpu.VMEM((BC + DM, KH), jnp.float32),
        ],
    )(xT, W_val, jnp.asarray(_CSA), jnp.asarray(_CSB), jnp.asarray(_CSC))

    out = pl.pallas_call(
        _fuse_body,
        grid=(NRB,),
        in_specs=[
            pl.BlockSpec((RB, NFP), lambda i: (i, 0)),
            pl.BlockSpec((RB, NFP), lambda i: (i, 0)),
            pl.BlockSpec((DM, NFP), lambda i: (0, 0)),
            pl.BlockSpec((DM, NFP), lambda i: (0, 0)),
            pl.BlockSpec((1, NFP), lambda i: (0, 0)),
            pl.BlockSpec((RB, DM), lambda i: (i, 0)),
            pl.BlockSpec((1, DM), lambda i: (0, 0)),
            pl.BlockSpec((DM, DM), lambda i: (0, 0)),
            pl.BlockSpec((DM, DM), lambda i: (0, 0)),
            pl.BlockSpec((1, DM), lambda i: (0, 0)),
            pl.BlockSpec((6, DM), lambda i: (0, 0)),
            pl.BlockSpec((4, DM), lambda i: (0, 0)),
            pl.BlockSpec((DFF, DM), lambda i: (0, 0)),
            pl.BlockSpec((1, DFF), lambda i: (0, 0)),
            pl.BlockSpec((6, DFF), lambda i: (0, 0)),
            pl.BlockSpec((4, DFF), lambda i: (0, 0)),
            pl.BlockSpec((DM, DFF), lambda i: (0, 0)),
            pl.BlockSpec((1, DM), lambda i: (0, 0)),
        ],
        out_specs=pl.BlockSpec((RB, DM), lambda i: (i, 0)),
        out_shape=jax.ShapeDtypeStruct((BC, DM), jnp.float32),
    )(xr, xs, wc, ws, jnp.asarray(_WVEC), jnp.asarray(_PE),
      glb.reshape(1, DM), W_emb[:, :DM], W_emb[:, DM:], b_emb.reshape(1, DM),
      jnp.repeat(a1, DM // G, axis=0).T, jnp.repeat(b1, DM // G, axis=0).T,
      W1, c1.reshape(1, DFF),
      jnp.repeat(a2, DFF // G, axis=0).T, jnp.repeat(b2, DFF // G, axis=0).T,
      W2, c2.reshape(1, DM))
    return out.reshape(B, C, DM)
